# baseline (speedup 1.0000x reference)
import jax
import jax.numpy as jnp
from jax import lax
from jax.experimental import pallas as pl
from jax.experimental.pallas import tpu as pltpu
from jax.experimental.pallas import tpu_sc as plsc

N = 10000
E = 320000
D = 128

NC = 2
NS = 16
NW = NC * NS
LANES = 16

CHUNK = 64
NCHUNK = E // CHUNK
BASE = NCHUNK // NW
REM = NCHUNK % NW
RPT = 640
NPAD = NS * RPT
NSLOT = 4


def _worker_span():
    cid = lax.axis_index("c")
    sid = lax.axis_index("s")
    wid = sid * NC + cid
    nw = BASE + jnp.where(wid < REM, 1, 0)
    a0 = wid * BASE + jnp.minimum(wid, REM)
    return cid, sid, a0, nw


def _unpack_lane(ed_v, src_row, src_off, dst_v):
    for j in range(CHUNK // LANES):
        dst_v[pl.ds(j * LANES, LANES)] = (
            ed_v[src_row, pl.ds(src_off + j * LANES, LANES)])


def _deg_body(ed_hbm, pd_hbm, edA, edB, col_v, ew_v, zb_v, deg_s, siA, siB):
    cid, sid, a0, nw = _worker_span()

    @pl.loop(0, RPT // LANES)
    def _(i):
        zb_v[pl.ds(i * LANES, LANES)] = jnp.zeros((LANES,), jnp.float32)

    pltpu.sync_copy(zb_v, deg_s.at[pl.ds(sid * RPT, RPT)])
    plsc.subcore_barrier()

    def _scatter(ed_v):
        _unpack_lane(ed_v, 1, 0, col_v)
        for j in range(CHUNK // LANES):
            ew_v[pl.ds(j * LANES, LANES)] = plsc.bitcast(
                ed_v[0, pl.ds(CHUNK + j * LANES, LANES)], jnp.float32)
        pltpu.sync_copy(ew_v, deg_s.at[col_v], add=True)

    pltpu.sync_copy(ed_hbm.at[a0], edA)

    @pl.when(nw > 1)
    def _():
        pltpu.async_copy(ed_hbm.at[a0 + 1], edB, siB)

    @pl.loop(0, (nw + 1) // 2)
    def _(p):
        i = 2 * p

        @pl.when(i > 0)
        def _():
            pltpu.make_async_copy(ed_hbm.at[0], edA, siA).wait()

        _scatter(edA)

        @pl.when(i + 2 < nw)
        def _():
            pltpu.async_copy(ed_hbm.at[a0 + i + 2], edA, siA)

        @pl.when(i + 1 < nw)
        def _():
            pltpu.make_async_copy(ed_hbm.at[0], edB, siB).wait()
            _scatter(edB)

            @pl.when(i + 3 < nw)
            def _():
                pltpu.async_copy(ed_hbm.at[a0 + i + 3], edB, siB)

    plsc.subcore_barrier()
    pltpu.sync_copy(deg_s.at[pl.ds(sid * RPT, RPT)],
                    pd_hbm.at[cid, pl.ds(sid * RPT, RPT)])


def _deg_partials(ed):
    return pl.kernel(
        _deg_body,
        out_type=jax.ShapeDtypeStruct((NC, NPAD), jnp.float32),
        mesh=plsc.VectorSubcoreMesh(core_axis_name="c", subcore_axis_name="s"),
        compiler_params=pltpu.CompilerParams(needs_layout_passes=False),
        scratch_types=[
            pltpu.VMEM((2, 2 * CHUNK), jnp.int32),
            pltpu.VMEM((2, 2 * CHUNK), jnp.int32),
            pltpu.VMEM((CHUNK,), jnp.int32),
            pltpu.VMEM((CHUNK,), jnp.float32),
            pltpu.VMEM((RPT,), jnp.float32),
            pltpu.VMEM_SHARED((NPAD,), jnp.float32),
            pltpu.SemaphoreType.DMA,
            pltpu.SemaphoreType.DMA,
        ],
    )(ed)


def _agg_body(g_hbm, ed_hbm, p_hbm, *args):
    ed = args[0:NSLOT]
    rowv = args[NSLOT:2 * NSLOT]
    colv = args[2 * NSLOT:3 * NSLOT]
    rows = args[3 * NSLOT:4 * NSLOT]
    sg = args[4 * NSLOT:5 * NSLOT]
    si = args[5 * NSLOT:6 * NSLOT]
    ss = args[6 * NSLOT:7 * NSLOT]
    acc_s = args[7 * NSLOT]
    cid, sid, a0, nw = _worker_span()

    @pl.loop(0, CHUNK)
    def _(i):
        for j in range(D // LANES):
            rows[0][i, pl.ds(j * LANES, LANES)] = jnp.zeros((LANES,),
                                                            jnp.float32)

    for k in range(RPT // CHUNK):
        pltpu.sync_copy(rows[0], acc_s.at[pl.ds(sid * RPT + k * CHUNK, CHUNK)])
    plsc.subcore_barrier()

    def _scale_fire(k):
        @pl.loop(0, CHUNK, unroll=4)
        def _(e):
            s = plsc.bitcast(
                plsc.load_gather(
                    ed[k], [jnp.zeros((LANES,), jnp.int32),
                            jnp.full((LANES,), CHUNK + e, jnp.int32)]),
                jnp.float32)
            for j in range(D // LANES):
                rows[k][e, pl.ds(j * LANES, LANES)] = (
                    rows[k][e, pl.ds(j * LANES, LANES)] * s)

        _unpack_lane(ed[k], 1, 0, colv[k])
        pltpu.async_copy(rows[k], acc_s.at[colv[k]], ss[k], add=True)

    for k in range(2):
        @pl.when(k < nw)
        def _(k=k):
            pltpu.sync_copy(ed_hbm.at[a0 + k], ed[k])
            _unpack_lane(ed[k], 0, 0, rowv[k])
            pltpu.async_copy(g_hbm.at[rowv[k]], rows[k], sg[k])

    for k in range(2, NSLOT):
        @pl.when(k < nw)
        def _(k=k):
            pltpu.async_copy(ed_hbm.at[a0 + k], ed[k], si[k])

    @pl.loop(0, (nw + NSLOT - 1) // NSLOT)
    def _(q):
        for k in range(NSLOT):
            c = NSLOT * q + k

            @pl.when(c < nw)
            def _(k=k, c=c):
                kn = (k + 2) % NSLOT
                pltpu.make_async_copy(g_hbm.at[rowv[k]], rows[k],
                                      sg[k]).wait()
                _scale_fire(k)

                @pl.when(c + NSLOT < nw)
                def _():
                    pltpu.async_copy(ed_hbm.at[a0 + c + NSLOT], ed[k], si[k])

                @pl.when(c + 2 < nw)
                def _():
                    @pl.when(c >= 2)
                    def _():
                        pltpu.make_async_copy(rows[kn], acc_s.at[colv[kn]],
                                              ss[kn]).wait()

                    pltpu.make_async_copy(ed_hbm.at[0], ed[kn], si[kn]).wait()
                    _unpack_lane(ed[kn], 0, 0, rowv[kn])
                    pltpu.async_copy(g_hbm.at[rowv[kn]], rows[kn], sg[kn])

    for k in range(NSLOT):
        @pl.when(k < nw)
        def _(k=k):
            pltpu.make_async_copy(rows[k], acc_s.at[colv[k]], ss[k]).wait()

    plsc.subcore_barrier()
    pltpu.sync_copy(acc_s.at[pl.ds(sid * RPT, RPT)],
                    p_hbm.at[cid, pl.ds(sid * RPT, RPT)])


def _aggregate(g, ed):
    return pl.kernel(
        _agg_body,
        out_type=jax.ShapeDtypeStruct((NC, NPAD, D), jnp.float32),
        mesh=plsc.VectorSubcoreMesh(core_axis_name="c", subcore_axis_name="s"),
        compiler_params=pltpu.CompilerParams(needs_layout_passes=False),
        scratch_types=(
            [pltpu.VMEM((2, 2 * CHUNK), jnp.int32)] * NSLOT
            + [pltpu.VMEM((CHUNK,), jnp.int32)] * NSLOT
            + [pltpu.VMEM((CHUNK,), jnp.int32)] * NSLOT
            + [pltpu.VMEM((CHUNK, D), jnp.float32)] * NSLOT
            + [pltpu.SemaphoreType.DMA] * (3 * NSLOT)
            + [pltpu.VMEM_SHARED((NPAD, D), jnp.float32)]
        ),
    )(g, ed)


def _lin_body(x_ref, w_ref, pd_ref, g_ref):
    deg = 1.0 + pd_ref[0] + pd_ref[1]
    dis = lax.rsqrt(deg)
    g_ref[...] = dis * jnp.dot(x_ref[...], w_ref[...],
                               preferred_element_type=jnp.float32)


def _linear(x, W, pd):
    blk = 400
    grid = N // blk
    return pl.pallas_call(
        _lin_body,
        grid=(grid,),
        in_specs=[
            pl.BlockSpec((blk, D), lambda i: (i, 0)),
            pl.BlockSpec((D, D), lambda i: (0, 0)),
            pl.BlockSpec((NC, blk, 1), lambda i: (0, i, 0)),
        ],
        out_specs=pl.BlockSpec((blk, D), lambda i: (i, 0)),
        out_shape=jax.ShapeDtypeStruct((N, D), jnp.float32),
    )(x, W, pd.reshape(NC, NPAD, 1))


def _fin_body(g_ref, p_ref, pd_ref, b_ref, o_ref):
    deg = 1.0 + pd_ref[0] + pd_ref[1]
    dis = lax.rsqrt(deg)
    o_ref[...] = dis * (g_ref[...] + p_ref[0] + p_ref[1]) + b_ref[...]


def _finalize(g, p, pd, b):
    blk = 400
    grid = N // blk
    return pl.pallas_call(
        _fin_body,
        grid=(grid,),
        in_specs=[
            pl.BlockSpec((blk, D), lambda i: (i, 0)),
            pl.BlockSpec((NC, blk, D), lambda i: (0, i, 0)),
            pl.BlockSpec((NC, blk, 1), lambda i: (0, i, 0)),
            pl.BlockSpec((1, D), lambda i: (0, 0)),
        ],
        out_specs=pl.BlockSpec((blk, D), lambda i: (i, 0)),
        out_shape=jax.ShapeDtypeStruct((N, D), jnp.float32),
    )(g, p, pd.reshape(NC, NPAD, 1), b.reshape(1, D))


def kernel(x, edge_index, edge_weight, W, b):
    row = edge_index[0]
    col = edge_index[1]
    r2 = row.reshape(NCHUNK, CHUNK)
    wb = lax.bitcast_convert_type(edge_weight, jnp.int32).reshape(NCHUNK, CHUNK)
    c2 = col.reshape(NCHUNK, CHUNK)
    ed = jnp.stack([jnp.concatenate([r2, wb], axis=1),
                    jnp.concatenate([c2, jnp.zeros_like(c2)], axis=1)], axis=1)
    pd = _deg_partials(ed)
    g = _linear(x, W, pd)
    p = _aggregate(g, ed)
    return _finalize(g, p, pd, b)


# speedup vs baseline: 1.0369x; 1.0369x over previous
import jax
import jax.numpy as jnp
from jax import lax
from jax.experimental import pallas as pl
from jax.experimental.pallas import tpu as pltpu
from jax.experimental.pallas import tpu_sc as plsc

N = 10000
E = 320000
D = 128

NC = 2
NS = 16
NW = NC * NS
LANES = 16

CHUNK = 128
NCHUNK = E // CHUNK
BASE = NCHUNK // NW
REM = NCHUNK % NW
RPT = 640
NPAD = NS * RPT


def _worker_span():
    cid = lax.axis_index("c")
    sid = lax.axis_index("s")
    wid = sid * NC + cid
    nw = BASE + jnp.where(wid < REM, 1, 0)
    a0 = wid * BASE + jnp.minimum(wid, REM)
    return cid, sid, a0, nw


def _unpack_lane(ed_v, src_row, dst_v):
    for j in range(CHUNK // LANES):
        dst_v[pl.ds(j * LANES, LANES)] = ed_v[src_row, pl.ds(j * LANES, LANES)]


def _deg_body(ed_hbm, pd_hbm, edA, edB, col_v, ew_v, zb_v, deg_s, siA, siB):
    cid, sid, a0, nw = _worker_span()

    @pl.loop(0, RPT // LANES)
    def _(i):
        zb_v[pl.ds(i * LANES, LANES)] = jnp.zeros((LANES,), jnp.float32)

    pltpu.sync_copy(zb_v, deg_s.at[pl.ds(sid * RPT, RPT)])
    plsc.subcore_barrier()

    def _scatter(ed_v):
        _unpack_lane(ed_v, 2, col_v)
        for j in range(CHUNK // LANES):
            ew_v[pl.ds(j * LANES, LANES)] = plsc.bitcast(
                ed_v[1, pl.ds(j * LANES, LANES)], jnp.float32)
        pltpu.sync_copy(ew_v, deg_s.at[col_v], add=True)

    pltpu.sync_copy(ed_hbm.at[a0], edA)

    @pl.when(nw > 1)
    def _():
        pltpu.async_copy(ed_hbm.at[a0 + 1], edB, siB)

    @pl.loop(0, (nw + 1) // 2)
    def _(p):
        i = 2 * p

        @pl.when(i > 0)
        def _():
            pltpu.make_async_copy(ed_hbm.at[0], edA, siA).wait()

        _scatter(edA)

        @pl.when(i + 2 < nw)
        def _():
            pltpu.async_copy(ed_hbm.at[a0 + i + 2], edA, siA)

        @pl.when(i + 1 < nw)
        def _():
            pltpu.make_async_copy(ed_hbm.at[0], edB, siB).wait()
            _scatter(edB)

            @pl.when(i + 3 < nw)
            def _():
                pltpu.async_copy(ed_hbm.at[a0 + i + 3], edB, siB)

    plsc.subcore_barrier()
    pltpu.sync_copy(deg_s.at[pl.ds(sid * RPT, RPT)],
                    pd_hbm.at[cid, pl.ds(sid * RPT, RPT)])


def _deg_partials(ed):
    return pl.kernel(
        _deg_body,
        out_type=jax.ShapeDtypeStruct((NC, NPAD), jnp.float32),
        mesh=plsc.VectorSubcoreMesh(core_axis_name="c", subcore_axis_name="s"),
        compiler_params=pltpu.CompilerParams(needs_layout_passes=False),
        scratch_types=[
            pltpu.VMEM((3, CHUNK), jnp.int32),
            pltpu.VMEM((3, CHUNK), jnp.int32),
            pltpu.VMEM((CHUNK,), jnp.int32),
            pltpu.VMEM((CHUNK,), jnp.float32),
            pltpu.VMEM((RPT,), jnp.float32),
            pltpu.VMEM_SHARED((NPAD,), jnp.float32),
            pltpu.SemaphoreType.DMA,
            pltpu.SemaphoreType.DMA,
        ],
    )(ed)


HALF = CHUNK // 2


def _agg_body(g_hbm, ed_hbm, p_hbm,
              edA, edB, rowA, rowB, colA1, colA2, colB1, colB2,
              rowsA, rowsB, acc_s,
              sgA, sgB, siA, siB, ssA, ssB):
    cid, sid, a0, nw = _worker_span()

    @pl.loop(0, CHUNK)
    def _(i):
        for j in range(D // LANES):
            rowsA[i, pl.ds(j * LANES, LANES)] = jnp.zeros((LANES,), jnp.float32)

    for k in range(RPT // CHUNK):
        pltpu.sync_copy(rowsA, acc_s.at[pl.ds(sid * RPT + k * CHUNK, CHUNK)])
    plsc.subcore_barrier()

    def _scale_half(ed_v, rows_v, h):
        @pl.loop(h * HALF, (h + 1) * HALF, unroll=4)
        def _(e):
            s = plsc.bitcast(
                plsc.load_gather(
                    ed_v, [jnp.ones((LANES,), jnp.int32),
                           jnp.full((LANES,), e, jnp.int32)]), jnp.float32)
            for j in range(D // LANES):
                rows_v[e, pl.ds(j * LANES, LANES)] = (
                    rows_v[e, pl.ds(j * LANES, LANES)] * s)

    def _scale_fire(ed_v, col_v1, col_v2, rows_v, sem):
        _scale_half(ed_v, rows_v, 0)
        for j in range(HALF // LANES):
            col_v1[pl.ds(j * LANES, LANES)] = ed_v[2, pl.ds(j * LANES, LANES)]
        pltpu.async_copy(rows_v.at[pl.ds(0, HALF)], acc_s.at[col_v1], sem,
                         add=True)
        _scale_half(ed_v, rows_v, 1)
        for j in range(HALF // LANES):
            col_v2[pl.ds(j * LANES, LANES)] = (
                ed_v[2, pl.ds(HALF + j * LANES, LANES)])
        pltpu.async_copy(rows_v.at[pl.ds(HALF, HALF)], acc_s.at[col_v2], sem,
                         add=True)

    def _drain(col_v1, col_v2, rows_v, sem):
        pltpu.make_async_copy(rows_v.at[pl.ds(0, HALF)], acc_s.at[col_v1],
                              sem).wait()
        pltpu.make_async_copy(rows_v.at[pl.ds(HALF, HALF)], acc_s.at[col_v2],
                              sem).wait()

    pltpu.sync_copy(ed_hbm.at[a0], edA)
    _unpack_lane(edA, 0, rowA)
    pltpu.async_copy(g_hbm.at[rowA], rowsA, sgA)

    @pl.when(nw > 1)
    def _():
        pltpu.async_copy(ed_hbm.at[a0 + 1], edB, siB)

    @pl.loop(0, (nw + 1) // 2)
    def _(p):
        i = 2 * p
        pltpu.make_async_copy(g_hbm.at[rowA], rowsA, sgA).wait()

        @pl.when(i + 1 < nw)
        def _():
            pltpu.make_async_copy(ed_hbm.at[0], edB, siB).wait()

            @pl.when(p > 0)
            def _():
                _drain(colB1, colB2, rowsB, ssB)

            _unpack_lane(edB, 0, rowB)
            pltpu.async_copy(g_hbm.at[rowB], rowsB, sgB)

        _scale_fire(edA, colA1, colA2, rowsA, ssA)

        @pl.when(i + 2 < nw)
        def _():
            pltpu.async_copy(ed_hbm.at[a0 + i + 2], edA, siA)

        @pl.when(i + 1 < nw)
        def _():
            pltpu.make_async_copy(g_hbm.at[rowB], rowsB, sgB).wait()
            _scale_fire(edB, colB1, colB2, rowsB, ssB)

            @pl.when(i + 3 < nw)
            def _():
                pltpu.async_copy(ed_hbm.at[a0 + i + 3], edB, siB)

        @pl.when(i + 2 < nw)
        def _():
            _drain(colA1, colA2, rowsA, ssA)
            pltpu.make_async_copy(ed_hbm.at[0], edA, siA).wait()
            _unpack_lane(edA, 0, rowA)
            pltpu.async_copy(g_hbm.at[rowA], rowsA, sgA)

    _drain(colA1, colA2, rowsA, ssA)

    @pl.when(nw > 1)
    def _():
        _drain(colB1, colB2, rowsB, ssB)

    plsc.subcore_barrier()
    pltpu.sync_copy(acc_s.at[pl.ds(sid * RPT, RPT)],
                    p_hbm.at[cid, pl.ds(sid * RPT, RPT)])


def _aggregate(g, ed):
    return pl.kernel(
        _agg_body,
        out_type=jax.ShapeDtypeStruct((NC, NPAD, D), jnp.float32),
        mesh=plsc.VectorSubcoreMesh(core_axis_name="c", subcore_axis_name="s"),
        compiler_params=pltpu.CompilerParams(needs_layout_passes=False),
        scratch_types=[
            pltpu.VMEM((3, CHUNK), jnp.int32),
            pltpu.VMEM((3, CHUNK), jnp.int32),
            pltpu.VMEM((CHUNK,), jnp.int32),
            pltpu.VMEM((CHUNK,), jnp.int32),
            pltpu.VMEM((HALF,), jnp.int32),
            pltpu.VMEM((HALF,), jnp.int32),
            pltpu.VMEM((HALF,), jnp.int32),
            pltpu.VMEM((HALF,), jnp.int32),
            pltpu.VMEM((CHUNK, D), jnp.float32),
            pltpu.VMEM((CHUNK, D), jnp.float32),
            pltpu.VMEM_SHARED((NPAD, D), jnp.float32),
            pltpu.SemaphoreType.DMA,
            pltpu.SemaphoreType.DMA,
            pltpu.SemaphoreType.DMA,
            pltpu.SemaphoreType.DMA,
            pltpu.SemaphoreType.DMA,
            pltpu.SemaphoreType.DMA,
        ],
    )(g, ed)


def _lin_body(x_ref, w_ref, pd_ref, g_ref):
    deg = 1.0 + pd_ref[0] + pd_ref[1]
    dis = lax.rsqrt(deg)
    g_ref[...] = dis * jnp.dot(x_ref[...], w_ref[...],
                               preferred_element_type=jnp.float32)


def _linear(x, W, pd):
    blk = 400
    grid = N // blk
    return pl.pallas_call(
        _lin_body,
        grid=(grid,),
        in_specs=[
            pl.BlockSpec((blk, D), lambda i: (i, 0)),
            pl.BlockSpec((D, D), lambda i: (0, 0)),
            pl.BlockSpec((NC, blk, 1), lambda i: (0, i, 0)),
        ],
        out_specs=pl.BlockSpec((blk, D), lambda i: (i, 0)),
        out_shape=jax.ShapeDtypeStruct((N, D), jnp.float32),
    )(x, W, pd.reshape(NC, NPAD, 1))


def _fin_body(g_ref, p_ref, pd_ref, b_ref, o_ref):
    deg = 1.0 + pd_ref[0] + pd_ref[1]
    dis = lax.rsqrt(deg)
    o_ref[...] = dis * (g_ref[...] + p_ref[0] + p_ref[1]) + b_ref[...]


def _finalize(g, p, pd, b):
    blk = 400
    grid = N // blk
    return pl.pallas_call(
        _fin_body,
        grid=(grid,),
        in_specs=[
            pl.BlockSpec((blk, D), lambda i: (i, 0)),
            pl.BlockSpec((NC, blk, D), lambda i: (0, i, 0)),
            pl.BlockSpec((NC, blk, 1), lambda i: (0, i, 0)),
            pl.BlockSpec((1, D), lambda i: (0, 0)),
        ],
        out_specs=pl.BlockSpec((blk, D), lambda i: (i, 0)),
        out_shape=jax.ShapeDtypeStruct((N, D), jnp.float32),
    )(g, p, pd.reshape(NC, NPAD, 1), b.reshape(1, D))


def kernel(x, edge_index, edge_weight, W, b):
    row = edge_index[0]
    col = edge_index[1]
    ed = jnp.stack(
        [row.reshape(NCHUNK, CHUNK),
         lax.bitcast_convert_type(edge_weight, jnp.int32).reshape(NCHUNK, CHUNK),
         col.reshape(NCHUNK, CHUNK)], axis=1)
    pd = _deg_partials(ed)
    g = _linear(x, W, pd)
    p = _aggregate(g, ed)
    return _finalize(g, p, pd, b)


# speedup vs baseline: 1.0473x; 1.0100x over previous
import jax
import jax.numpy as jnp
from jax import lax
from jax.experimental import pallas as pl
from jax.experimental.pallas import tpu as pltpu
from jax.experimental.pallas import tpu_sc as plsc

N = 10000
E = 320000
D = 128

NC = 2
NS = 16
NW = NC * NS
LANES = 16

CHUNK = 128
NCHUNK = E // CHUNK
BASE = NCHUNK // NW
REM = NCHUNK % NW
RPT = 640
NPAD = NS * RPT


def _worker_span():
    cid = lax.axis_index("c")
    sid = lax.axis_index("s")
    wid = sid * NC + cid
    nw = BASE + jnp.where(wid < REM, 1, 0)
    a0 = wid * BASE + jnp.minimum(wid, REM)
    return cid, sid, a0, nw


def _unpack_lane(ed_v, src_row, dst_v):
    for j in range(CHUNK // LANES):
        dst_v[pl.ds(j * LANES, LANES)] = ed_v[src_row, pl.ds(j * LANES, LANES)]


def _deg_body(ed_hbm, pd_hbm, edA, edB, col_v, ew_v, zb_v, deg_s, siA, siB):
    cid, sid, a0, nw = _worker_span()

    @pl.loop(0, RPT // LANES)
    def _(i):
        zb_v[pl.ds(i * LANES, LANES)] = jnp.zeros((LANES,), jnp.float32)

    pltpu.sync_copy(zb_v, deg_s.at[pl.ds(sid * RPT, RPT)])
    plsc.subcore_barrier()

    def _scatter(ed_v):
        _unpack_lane(ed_v, 2, col_v)
        for j in range(CHUNK // LANES):
            ew_v[pl.ds(j * LANES, LANES)] = plsc.bitcast(
                ed_v[1, pl.ds(j * LANES, LANES)], jnp.float32)
        pltpu.sync_copy(ew_v, deg_s.at[col_v], add=True)

    pltpu.sync_copy(ed_hbm.at[a0], edA)

    @pl.when(nw > 1)
    def _():
        pltpu.async_copy(ed_hbm.at[a0 + 1], edB, siB)

    @pl.loop(0, (nw + 1) // 2)
    def _(p):
        i = 2 * p

        @pl.when(i > 0)
        def _():
            pltpu.make_async_copy(ed_hbm.at[0], edA, siA).wait()

        _scatter(edA)

        @pl.when(i + 2 < nw)
        def _():
            pltpu.async_copy(ed_hbm.at[a0 + i + 2], edA, siA)

        @pl.when(i + 1 < nw)
        def _():
            pltpu.make_async_copy(ed_hbm.at[0], edB, siB).wait()
            _scatter(edB)

            @pl.when(i + 3 < nw)
            def _():
                pltpu.async_copy(ed_hbm.at[a0 + i + 3], edB, siB)

    plsc.subcore_barrier()
    pltpu.sync_copy(deg_s.at[pl.ds(sid * RPT, RPT)],
                    pd_hbm.at[cid, pl.ds(sid * RPT, RPT)])


def _deg_partials(ed):
    return pl.kernel(
        _deg_body,
        out_type=jax.ShapeDtypeStruct((NC, NPAD), jnp.float32),
        mesh=plsc.VectorSubcoreMesh(core_axis_name="c", subcore_axis_name="s"),
        compiler_params=pltpu.CompilerParams(needs_layout_passes=False),
        scratch_types=[
            pltpu.VMEM((3, CHUNK), jnp.int32),
            pltpu.VMEM((3, CHUNK), jnp.int32),
            pltpu.VMEM((CHUNK,), jnp.int32),
            pltpu.VMEM((CHUNK,), jnp.float32),
            pltpu.VMEM((RPT,), jnp.float32),
            pltpu.VMEM_SHARED((NPAD,), jnp.float32),
            pltpu.SemaphoreType.DMA,
            pltpu.SemaphoreType.DMA,
        ],
    )(ed)


def _agg_body(g_hbm, ed_hbm, p_hbm,
              edA, edB, rowA, rowB, colA, colB, rowsA, rowsB, acc_s,
              sgA, sgB, siA, siB, ssA, ssB):
    cid, sid, a0, nw = _worker_span()

    @pl.loop(0, CHUNK)
    def _(i):
        for j in range(D // LANES):
            rowsA[i, pl.ds(j * LANES, LANES)] = jnp.zeros((LANES,), jnp.float32)

    for k in range(RPT // CHUNK):
        pltpu.sync_copy(rowsA, acc_s.at[pl.ds(sid * RPT + k * CHUNK, CHUNK)])
    plsc.subcore_barrier()

    def _scale_fire(ed_v, col_v, rows_v, sem):
        @pl.loop(0, CHUNK, unroll=4)
        def _(e):
            s = plsc.bitcast(
                plsc.load_gather(
                    ed_v, [jnp.ones((LANES,), jnp.int32),
                           jnp.full((LANES,), e, jnp.int32)]), jnp.float32)
            for j in range(D // LANES):
                rows_v[e, pl.ds(j * LANES, LANES)] = (
                    rows_v[e, pl.ds(j * LANES, LANES)] * s)

        _unpack_lane(ed_v, 2, col_v)
        pltpu.async_copy(rows_v, acc_s.at[col_v], sem, add=True)

    pltpu.sync_copy(ed_hbm.at[a0], edA)
    _unpack_lane(edA, 0, rowA)
    pltpu.async_copy(g_hbm.at[rowA], rowsA, sgA)

    @pl.when(nw > 1)
    def _():
        pltpu.async_copy(ed_hbm.at[a0 + 1], edB, siB)

    @pl.loop(0, (nw + 1) // 2)
    def _(p):
        i = 2 * p
        pltpu.make_async_copy(g_hbm.at[rowA], rowsA, sgA).wait()

        @pl.when(i + 1 < nw)
        def _():
            pltpu.make_async_copy(ed_hbm.at[0], edB, siB).wait()

            @pl.when(p > 0)
            def _():
                pltpu.make_async_copy(rowsB, acc_s.at[colB], ssB).wait()

            _unpack_lane(edB, 0, rowB)
            pltpu.async_copy(g_hbm.at[rowB], rowsB, sgB)

        _scale_fire(edA, colA, rowsA, ssA)

        @pl.when(i + 2 < nw)
        def _():
            pltpu.async_copy(ed_hbm.at[a0 + i + 2], edA, siA)

        @pl.when(i + 1 < nw)
        def _():
            pltpu.make_async_copy(g_hbm.at[rowB], rowsB, sgB).wait()
            _scale_fire(edB, colB, rowsB, ssB)

            @pl.when(i + 3 < nw)
            def _():
                pltpu.async_copy(ed_hbm.at[a0 + i + 3], edB, siB)

        @pl.when(i + 2 < nw)
        def _():
            pltpu.make_async_copy(rowsA, acc_s.at[colA], ssA).wait()
            pltpu.make_async_copy(ed_hbm.at[0], edA, siA).wait()
            _unpack_lane(edA, 0, rowA)
            pltpu.async_copy(g_hbm.at[rowA], rowsA, sgA)

    pltpu.make_async_copy(rowsA, acc_s.at[colA], ssA).wait()

    @pl.when(nw > 1)
    def _():
        pltpu.make_async_copy(rowsB, acc_s.at[colB], ssB).wait()

    plsc.subcore_barrier()
    pltpu.sync_copy(acc_s.at[pl.ds(sid * RPT, RPT)],
                    p_hbm.at[cid, pl.ds(sid * RPT, RPT)])


def _aggregate(g, ed):
    return pl.kernel(
        _agg_body,
        out_type=jax.ShapeDtypeStruct((NC, NPAD, D), jnp.float32),
        mesh=plsc.VectorSubcoreMesh(core_axis_name="c", subcore_axis_name="s"),
        compiler_params=pltpu.CompilerParams(needs_layout_passes=False),
        scratch_types=[
            pltpu.VMEM((3, CHUNK), jnp.int32),
            pltpu.VMEM((3, CHUNK), jnp.int32),
            pltpu.VMEM((CHUNK,), jnp.int32),
            pltpu.VMEM((CHUNK,), jnp.int32),
            pltpu.VMEM((CHUNK,), jnp.int32),
            pltpu.VMEM((CHUNK,), jnp.int32),
            pltpu.VMEM((CHUNK, D), jnp.float32),
            pltpu.VMEM((CHUNK, D), jnp.float32),
            pltpu.VMEM_SHARED((NPAD, D), jnp.float32),
            pltpu.SemaphoreType.DMA,
            pltpu.SemaphoreType.DMA,
            pltpu.SemaphoreType.DMA,
            pltpu.SemaphoreType.DMA,
            pltpu.SemaphoreType.DMA,
            pltpu.SemaphoreType.DMA,
        ],
    )(g, ed)


def _mm_body(x_ref, w_ref, h_ref):
    h_ref[...] = jnp.dot(x_ref[...], w_ref[...],
                         preferred_element_type=jnp.float32)


def _matmul(x, W):
    blk = 400
    grid = N // blk
    return pl.pallas_call(
        _mm_body,
        grid=(grid,),
        in_specs=[
            pl.BlockSpec((blk, D), lambda i: (i, 0)),
            pl.BlockSpec((D, D), lambda i: (0, 0)),
        ],
        out_specs=pl.BlockSpec((blk, D), lambda i: (i, 0)),
        out_shape=jax.ShapeDtypeStruct((N, D), jnp.float32),
    )(x, W)


def _scale_body(h_ref, pd_ref, g_ref):
    deg = 1.0 + pd_ref[0] + pd_ref[1]
    dis = lax.rsqrt(deg)
    g_ref[...] = dis * h_ref[...]


def _scale_g(h, pd):
    blk = 400
    grid = N // blk
    return pl.pallas_call(
        _scale_body,
        grid=(grid,),
        in_specs=[
            pl.BlockSpec((blk, D), lambda i: (i, 0)),
            pl.BlockSpec((NC, blk, 1), lambda i: (0, i, 0)),
        ],
        out_specs=pl.BlockSpec((blk, D), lambda i: (i, 0)),
        out_shape=jax.ShapeDtypeStruct((N, D), jnp.float32),
    )(h, pd.reshape(NC, NPAD, 1))


def _fin_body(g_ref, p_ref, pd_ref, b_ref, o_ref):
    deg = 1.0 + pd_ref[0] + pd_ref[1]
    dis = lax.rsqrt(deg)
    o_ref[...] = dis * (g_ref[...] + p_ref[0] + p_ref[1]) + b_ref[...]


def _finalize(g, p, pd, b):
    blk = 400
    grid = N // blk
    return pl.pallas_call(
        _fin_body,
        grid=(grid,),
        in_specs=[
            pl.BlockSpec((blk, D), lambda i: (i, 0)),
            pl.BlockSpec((NC, blk, D), lambda i: (0, i, 0)),
            pl.BlockSpec((NC, blk, 1), lambda i: (0, i, 0)),
            pl.BlockSpec((1, D), lambda i: (0, 0)),
        ],
        out_specs=pl.BlockSpec((blk, D), lambda i: (i, 0)),
        out_shape=jax.ShapeDtypeStruct((N, D), jnp.float32),
    )(g, p, pd.reshape(NC, NPAD, 1), b.reshape(1, D))


def kernel(x, edge_index, edge_weight, W, b):
    row = edge_index[0]
    col = edge_index[1]
    ed = jnp.stack(
        [row.reshape(NCHUNK, CHUNK),
         lax.bitcast_convert_type(edge_weight, jnp.int32).reshape(NCHUNK, CHUNK),
         col.reshape(NCHUNK, CHUNK)], axis=1)
    h = _matmul(x, W)
    pd = _deg_partials(ed)
    g = _scale_g(h, pd)
    p = _aggregate(g, ed)
    return _finalize(g, p, pd, b)
